# baseline (speedup 1.0000x reference)
import functools

import jax
import jax.numpy as jnp
import numpy as np
from jax import lax
from jax.experimental import pallas as pl
from jax.experimental.pallas import tpu as pltpu
from jax.experimental.pallas import tpu_sc as plsc

NUM_NEG = 10
NCOL = NUM_NEG + 1
NCOL_OUT = 16
TEMP = 0.1

NC = 2
NS = 16
NW = NC * NS

BLK = 32
NSLOT = 3


def _sc_worker_id():
    return lax.axis_index("s") * NC + lax.axis_index("c")


def _tf2x32(k0, k1, x0, x1):
    rot0, rot1 = (13, 15, 26, 6), (17, 29, 16, 24)
    u32 = np.uint32
    ks0, ks1 = u32(k0), u32(k1)
    ks2 = ks0 ^ ks1 ^ u32(0x1BD11BDA)
    x0 = (x0 + ks0).astype(u32)
    x1 = (x1 + ks1).astype(u32)

    def rounds(x0, x1, rots):
        for r in rots:
            x0 = (x0 + x1).astype(u32)
            x1 = ((x1 << u32(r)) | (x1 >> u32(32 - r))).astype(u32)
            x1 = x0 ^ x1
        return x0, x1

    for i, (rots, ka, kb) in enumerate(
        [(rot0, ks1, ks2), (rot1, ks2, ks0), (rot0, ks0, ks1),
         (rot1, ks1, ks2), (rot0, ks2, ks0)]
    ):
        x0, x1 = rounds(x0, x1, rots)
        x0 = (x0 + ka).astype(u32)
        x1 = (x1 + kb + u32(i + 1)).astype(u32)
    return x0, x1


def _np_randint(key01, k, n, m, span):
    u32 = np.uint32
    k0, k1 = key01
    f0, f1 = _tf2x32(k0, k1, np.array([0], u32), np.array([k], u32))
    kf0, kf1 = f0[0], f1[0]
    b1, b2 = _tf2x32(kf0, kf1, np.zeros(2, u32), np.arange(2, dtype=u32))
    sub = [(b1[0], b2[0]), (b1[1], b2[1])]
    size = n * m
    i = np.arange(size, dtype=np.uint64)
    c1 = (i >> np.uint64(32)).astype(u32)
    c2 = i.astype(u32)
    o0, o1 = _tf2x32(sub[0][0], sub[0][1], c1, c2)
    hi = (o0 ^ o1).reshape(n, m)
    o0, o1 = _tf2x32(sub[1][0], sub[1][1], c1, c2)
    lo = (o0 ^ o1).reshape(n, m)
    span_u = u32(span)
    mult = u32(np.uint64(65536) % np.uint64(span))
    mult = u32((np.uint64(mult) * np.uint64(mult)) % np.uint64(span))
    val = ((hi % span_u) * mult + (lo % span_u)) % span_u
    return val.astype(np.int32)


@functools.lru_cache(maxsize=None)
def _constants(B, C, S, K):
    idx = np.zeros((K, B, S, NCOL), np.int32)
    w = np.zeros((K, B, S, NCOL_OUT), np.float32)
    for k in range(1, K + 1):
        L = S - k
        npos = B * L
        b = np.arange(B, dtype=np.int64)[:, None]
        i = np.arange(L, dtype=np.int64)[None, :]
        idx[k - 1, :, :L, 0] = (b * S + i + k).astype(np.int32)
        idx[k - 1, :, :L, 1:] = _np_randint(
            (np.uint32(0), np.uint32(1234)), k, npos, NUM_NEG, B * S
        ).reshape(B, L, NUM_NEG)
        w[k - 1, :, :L, 0] = 1.0 / (K * npos)
        w[k - 1, :, :L, 1:NCOL] = 1.0 / (K * npos * NUM_NEG)
    P_pad = K * B * S
    assert P_pad % (NW * BLK) == 0
    return idx.reshape(-1), w.reshape(-1), P_pad


def _sc_sims(z_flat, cp_all, idx_flat, P_pad):
    samp_per_tile = P_pad // NW
    nblk = samp_per_tile // BLK
    ipb = BLK * NCOL
    C = cp_all.shape[1]
    zw = z_flat.shape[1]
    nc32 = C // 32
    chunks = []
    off = 0
    while off < ipb:
        n = min(128, ipb - off)
        chunks.append((off, n))
        off += n

    mesh = plsc.VectorSubcoreMesh(
        core_axis_name="c", subcore_axis_name="s", num_cores=NC, num_subcores=NS
    )

    @functools.partial(
        pl.kernel,
        out_type=jax.ShapeDtypeStruct((P_pad * NCOL_OUT,), jnp.float32),
        mesh=mesh,
        compiler_params=pltpu.CompilerParams(
            needs_layout_passes=False, use_tc_tiling_on_sc=False
        ),
        scratch_types=dict(
            idx_v=[[pltpu.VMEM((n,), jnp.int32) for _, n in chunks] for _ in range(NSLOT)],
            z_v=[pltpu.VMEM((ipb, zw), jnp.float32) for _ in range(NSLOT)],
            cp_v=[pltpu.VMEM((BLK, C), jnp.bfloat16) for _ in range(NSLOT)],
            sims_v=[pltpu.VMEM((BLK * NCOL_OUT,), jnp.float32) for _ in range(NSLOT)],
            tr_v=pltpu.VMEM((2 * NCOL_OUT * 16,), jnp.float32),
            sem_idx=[pltpu.SemaphoreType.DMA for _ in range(NSLOT)],
            sem_g=[pltpu.SemaphoreType.DMA for _ in range(NSLOT)],
            sem_cp=[pltpu.SemaphoreType.DMA for _ in range(NSLOT)],
            sem_wb=[pltpu.SemaphoreType.DMA for _ in range(NSLOT)],
        ),
    )
    def sc_kernel(z_hbm, cp_hbm, idx_hbm, out_hbm, *, idx_v, z_v, cp_v, sims_v,
                  tr_v, sem_idx, sem_g, sem_cp, sem_wb):
        wid = _sc_worker_id()
        samp0 = wid * samp_per_tile

        for tb in (0, NCOL_OUT * 16):
            for r in range(NCOL, NCOL_OUT):
                tr_v[pl.ds(tb + r * 16, 16)] = jnp.zeros((16,), jnp.float32)

        def idx_copies(g, s):
            return [
                pltpu.make_async_copy(
                    idx_hbm.at[pl.ds((samp0 + g * BLK) * NCOL + off, n)],
                    idx_v[s][ci],
                    sem_idx[s],
                )
                for ci, (off, n) in enumerate(chunks)
            ]

        def cp_copy(g, s):
            return pltpu.make_async_copy(
                cp_hbm.at[pl.ds(samp0 + g * BLK, BLK)], cp_v[s], sem_cp[s]
            )

        def gather_copies(s):
            return [
                pltpu.make_async_copy(
                    z_hbm.at[idx_v[s][ci]],
                    z_v[s].at[pl.ds(off, n)],
                    sem_g[s],
                )
                for ci, (off, n) in enumerate(chunks)
            ]

        def wb_copy(g, s):
            return pltpu.make_async_copy(
                sims_v[s],
                out_hbm.at[pl.ds((samp0 + g * BLK) * NCOL_OUT, BLK * NCOL_OUT)],
                sem_wb[s],
            )

        def stage_a_idx(g, s):
            for cpy in idx_copies(g, s):
                cpy.start()

        def stage_a_cp(g, s):
            cp_copy(g, s).start()

        def stage_b(g, s):
            for cpy in idx_copies(g, s):
                cpy.wait()
            for cpy in gather_copies(s):
                cpy.start()

        def stage_c_wait(g, s):
            for cpy in gather_copies(s):
                cpy.wait()
            cp_copy(g, s).wait()

            @pl.when(g >= NSLOT)
            def _():
                wb_copy(g - NSLOT, s).wait()

        def stage_c_comp(g, s):

            lanes = lax.iota(jnp.int32, 16)
            sign = jnp.where(lanes == 0, -1.0, 1.0).astype(jnp.float32)
            gather_ids = {
                tb: [lanes * 16 + (tb + c) for c in range(16)]
                for tb in (0, NCOL_OUT * 16)
            }

            def one_sample(i, tb):
                cpv = []
                for c32 in range(nc32):
                    cpv.extend(
                        plsc.unpack(
                            cp_v[s][i, pl.ds(c32 * 32, 32)],
                            format=plsc.PackFormat.INTERLEAVED,
                            preferred_element_type=jnp.float32,
                        )
                    )
                for r in range(NCOL):
                    row = i * NCOL + r
                    ts = []
                    for c32 in range(nc32):
                        za, zb = plsc.unpack(
                            plsc.bitcast(
                                z_v[s][row, pl.ds(c32 * 16, 16)], jnp.bfloat16
                            ),
                            format=plsc.PackFormat.INTERLEAVED,
                            preferred_element_type=jnp.float32,
                        )
                        ts.append(za * cpv[2 * c32] + zb * cpv[2 * c32 + 1])
                    while len(ts) > 1:
                        ts = [
                            ts[j] + ts[j + 1] if j + 1 < len(ts) else ts[j]
                            for j in range(0, len(ts), 2)
                        ]
                    tr_v[pl.ds(tb + r * 16, 16)] = ts[0]
                gs = [plsc.load_gather(tr_v, [gid]) for gid in gather_ids[tb]]
                while len(gs) > 1:
                    gs = [
                        gs[j] + gs[j + 1] if j + 1 < len(gs) else gs[j]
                        for j in range(0, len(gs), 2)
                    ]
                sims_v[s][pl.ds(i * NCOL_OUT, NCOL_OUT)] = gs[0] * sign

            @pl.loop(0, BLK, step=2)
            def _(i):
                one_sample(i, 0)
                one_sample(i + 1, NCOL_OUT * 16)

            wb_copy(g, s).start()

        for g in range(min(NSLOT, nblk)):
            stage_a_idx(g, g % NSLOT)
            stage_a_cp(g, g % NSLOT)
        for g in range(min(NSLOT - 1, nblk)):
            stage_b(g, g % NSLOT)

        @pl.loop(0, nblk, step=NSLOT)
        def _(g0):
            for dg in range(NSLOT):
                g = g0 + dg
                s = dg

                @pl.when(g + NSLOT - 1 < nblk)
                def _():
                    stage_b(g + NSLOT - 1, (dg + NSLOT - 1) % NSLOT)

                @pl.when(g < nblk)
                def _():
                    stage_c_wait(g, s)

                @pl.when(g + NSLOT < nblk)
                def _():
                    stage_a_idx(g + NSLOT, s)

                @pl.when(g < nblk)
                def _():
                    stage_c_comp(g, s)

                @pl.when(g + NSLOT < nblk)
                def _():
                    stage_a_cp(g + NSLOT, s)

        for t in range(NSLOT):
            g = nblk - NSLOT + t
            wb_copy(g, g % NSLOT).wait()

    return sc_kernel(z_flat, cp_all, idx_flat)


def _tc_reduce(sims2d, w2d):
    rows = sims2d.shape[0]
    br = 2048
    grid = rows // br

    def body(a_ref, w_ref, o_ref):
        @pl.when(pl.program_id(0) == 0)
        def _():
            o_ref[...] = jnp.zeros_like(o_ref)

        x = a_ref[...]
        sp = jnp.maximum(x, 0.0) + jnp.log1p(jnp.exp(-jnp.abs(x)))
        o_ref[...] += jnp.sum(w_ref[...] * sp).reshape(1, 1)

    out = pl.pallas_call(
        body,
        grid=(grid,),
        in_specs=[
            pl.BlockSpec((br, 128), lambda i: (i, 0)),
            pl.BlockSpec((br, 128), lambda i: (i, 0)),
        ],
        out_specs=pl.BlockSpec((1, 1), lambda i: (0, 0)),
        out_shape=jax.ShapeDtypeStruct((1, 1), jnp.float32),
    )(sims2d, w2d)
    return out[0, 0]


def kernel(z, c, predictions):
    del c
    B, C, S = z.shape
    K = predictions.shape[0]
    idx_flat, w_flat, P_pad = _constants(B, C, S, K)

    z_flat = jnp.transpose(z, (0, 2, 1)).reshape(-1, C).astype(jnp.bfloat16)
    z_flat = jax.lax.bitcast_convert_type(
        z_flat.reshape(-1, C // 2, 2), jnp.float32
    )
    cp_all = (
        jnp.transpose(predictions, (0, 1, 3, 2)).reshape(-1, C)
        * jnp.float32(1.0 / TEMP)
    ).astype(jnp.bfloat16)

    sims = _sc_sims(z_flat, cp_all, jnp.asarray(idx_flat), P_pad)
    sims2d = sims.reshape(-1, 128)
    w2d = jnp.asarray(w_flat).reshape(-1, 128)
    return _tc_reduce(sims2d, w2d)


# speedup vs baseline: 1.0057x; 1.0057x over previous
import functools

import jax
import jax.numpy as jnp
import numpy as np
from jax import lax
from jax.experimental import pallas as pl
from jax.experimental.pallas import tpu as pltpu
from jax.experimental.pallas import tpu_sc as plsc

NUM_NEG = 10
NCOL = NUM_NEG + 1
NCOL_OUT = 16
TEMP = 0.1

NC = 2
NS = 16
NW = NC * NS

BLK = 32
NSLOT = 3


def _sc_worker_id():
    return lax.axis_index("s") * NC + lax.axis_index("c")


def _tf2x32(k0, k1, x0, x1):
    rot0, rot1 = (13, 15, 26, 6), (17, 29, 16, 24)
    u32 = np.uint32
    ks0, ks1 = u32(k0), u32(k1)
    ks2 = ks0 ^ ks1 ^ u32(0x1BD11BDA)
    x0 = (x0 + ks0).astype(u32)
    x1 = (x1 + ks1).astype(u32)

    def rounds(x0, x1, rots):
        for r in rots:
            x0 = (x0 + x1).astype(u32)
            x1 = ((x1 << u32(r)) | (x1 >> u32(32 - r))).astype(u32)
            x1 = x0 ^ x1
        return x0, x1

    for i, (rots, ka, kb) in enumerate(
        [(rot0, ks1, ks2), (rot1, ks2, ks0), (rot0, ks0, ks1),
         (rot1, ks1, ks2), (rot0, ks2, ks0)]
    ):
        x0, x1 = rounds(x0, x1, rots)
        x0 = (x0 + ka).astype(u32)
        x1 = (x1 + kb + u32(i + 1)).astype(u32)
    return x0, x1


def _np_randint(key01, k, n, m, span):
    u32 = np.uint32
    k0, k1 = key01
    f0, f1 = _tf2x32(k0, k1, np.array([0], u32), np.array([k], u32))
    kf0, kf1 = f0[0], f1[0]
    b1, b2 = _tf2x32(kf0, kf1, np.zeros(2, u32), np.arange(2, dtype=u32))
    sub = [(b1[0], b2[0]), (b1[1], b2[1])]
    size = n * m
    i = np.arange(size, dtype=np.uint64)
    c1 = (i >> np.uint64(32)).astype(u32)
    c2 = i.astype(u32)
    o0, o1 = _tf2x32(sub[0][0], sub[0][1], c1, c2)
    hi = (o0 ^ o1).reshape(n, m)
    o0, o1 = _tf2x32(sub[1][0], sub[1][1], c1, c2)
    lo = (o0 ^ o1).reshape(n, m)
    span_u = u32(span)
    mult = u32(np.uint64(65536) % np.uint64(span))
    mult = u32((np.uint64(mult) * np.uint64(mult)) % np.uint64(span))
    val = ((hi % span_u) * mult + (lo % span_u)) % span_u
    return val.astype(np.int32)


@functools.lru_cache(maxsize=None)
def _constants(B, C, S, K):
    idx = np.zeros((K, B, S, NCOL), np.int32)
    w = np.zeros((K, B, S, NCOL_OUT), np.float32)
    for k in range(1, K + 1):
        L = S - k
        npos = B * L
        b = np.arange(B, dtype=np.int64)[:, None]
        i = np.arange(L, dtype=np.int64)[None, :]
        idx[k - 1, :, :L, 0] = (b * S + i + k).astype(np.int32)
        idx[k - 1, :, :L, 1:] = _np_randint(
            (np.uint32(0), np.uint32(1234)), k, npos, NUM_NEG, B * S
        ).reshape(B, L, NUM_NEG)
        w[k - 1, :, :L, 0] = 1.0 / (K * npos)
        w[k - 1, :, :L, 1:NCOL] = 1.0 / (K * npos * NUM_NEG)
    P_pad = K * B * S
    assert P_pad % (NW * BLK) == 0
    return idx.reshape(-1), w.reshape(-1), P_pad


def _sc_sims(z_flat, cp_all, idx_flat, P_pad):
    samp_per_tile = P_pad // NW
    nblk = samp_per_tile // BLK
    ipb = BLK * NCOL
    C = cp_all.shape[1]
    zw = z_flat.shape[1]
    nc32 = C // 32
    chunks = []
    off = 0
    while off < ipb:
        n = min(128, ipb - off)
        chunks.append((off, n))
        off += n

    mesh = plsc.VectorSubcoreMesh(
        core_axis_name="c", subcore_axis_name="s", num_cores=NC, num_subcores=NS
    )

    @functools.partial(
        pl.kernel,
        out_type=jax.ShapeDtypeStruct((P_pad * NCOL_OUT,), jnp.float32),
        mesh=mesh,
        compiler_params=pltpu.CompilerParams(
            needs_layout_passes=False, use_tc_tiling_on_sc=False
        ),
        scratch_types=dict(
            idx_v=[[pltpu.VMEM((n,), jnp.int32) for _, n in chunks] for _ in range(NSLOT)],
            z_v=[pltpu.VMEM((ipb, zw), jnp.float32) for _ in range(NSLOT)],
            cp_v=[pltpu.VMEM((BLK, C), jnp.bfloat16) for _ in range(NSLOT)],
            sims_v=[pltpu.VMEM((BLK * NCOL_OUT,), jnp.float32) for _ in range(NSLOT)],
            tr_v=pltpu.VMEM((2 * NCOL_OUT * 16,), jnp.float32),
            sem_idx=[pltpu.SemaphoreType.DMA for _ in range(NSLOT)],
            sem_g=[pltpu.SemaphoreType.DMA for _ in range(NSLOT)],
            sem_cp=[pltpu.SemaphoreType.DMA for _ in range(NSLOT)],
            sem_wb=[pltpu.SemaphoreType.DMA for _ in range(NSLOT)],
        ),
    )
    def sc_kernel(z_hbm, cp_hbm, idx_hbm, out_hbm, *, idx_v, z_v, cp_v, sims_v,
                  tr_v, sem_idx, sem_g, sem_cp, sem_wb):
        wid = _sc_worker_id()
        samp0 = wid * samp_per_tile

        for tb in (0, NCOL_OUT * 16):
            for r in range(NCOL, NCOL_OUT):
                tr_v[pl.ds(tb + r * 16, 16)] = jnp.zeros((16,), jnp.float32)

        def idx_copies(g, s):
            return [
                pltpu.make_async_copy(
                    idx_hbm.at[pl.ds((samp0 + g * BLK) * NCOL + off, n)],
                    idx_v[s][ci],
                    sem_idx[s],
                )
                for ci, (off, n) in enumerate(chunks)
            ]

        def cp_copy(g, s):
            return pltpu.make_async_copy(
                cp_hbm.at[pl.ds(samp0 + g * BLK, BLK)], cp_v[s], sem_cp[s]
            )

        def gather_copies(s):
            return [
                pltpu.make_async_copy(
                    z_hbm.at[idx_v[s][ci]],
                    z_v[s].at[pl.ds(off, n)],
                    sem_g[s],
                )
                for ci, (off, n) in enumerate(chunks)
            ]

        def wb_copy(g, s):
            return pltpu.make_async_copy(
                sims_v[s],
                out_hbm.at[pl.ds((samp0 + g * BLK) * NCOL_OUT, BLK * NCOL_OUT)],
                sem_wb[s],
            )

        def stage_a_idx(g, s):
            for cpy in idx_copies(g, s):
                cpy.start()

        def stage_a_cp(g, s):
            cp_copy(g, s).start()

        def stage_b(g, s):
            for cpy in idx_copies(g, s):
                cpy.wait()
            for cpy in gather_copies(s):
                cpy.start()

        def stage_c_wait(g, s):
            for cpy in gather_copies(s):
                cpy.wait()
            cp_copy(g, s).wait()

            @pl.when(g >= NSLOT)
            def _():
                wb_copy(g - NSLOT, s).wait()

        def stage_c_comp(g, s):

            lanes = lax.iota(jnp.int32, 16)
            sign = jnp.where(lanes == 0, -1.0, 1.0).astype(jnp.float32)
            gather_ids = {
                tb: [lanes * 16 + (tb + c) for c in range(16)]
                for tb in (0, NCOL_OUT * 16)
            }

            def one_sample(i, tb):
                cpv = []
                for c32 in range(nc32):
                    cpv.extend(
                        plsc.unpack(
                            cp_v[s][i, pl.ds(c32 * 32, 32)],
                            format=plsc.PackFormat.INTERLEAVED,
                            preferred_element_type=jnp.float32,
                        )
                    )
                for r in range(NCOL):
                    row = i * NCOL + r
                    ts = []
                    for c32 in range(nc32):
                        za, zb = plsc.unpack(
                            plsc.bitcast(
                                z_v[s][row, pl.ds(c32 * 16, 16)], jnp.bfloat16
                            ),
                            format=plsc.PackFormat.INTERLEAVED,
                            preferred_element_type=jnp.float32,
                        )
                        ts.append(za * cpv[2 * c32] + zb * cpv[2 * c32 + 1])
                    while len(ts) > 1:
                        ts = [
                            ts[j] + ts[j + 1] if j + 1 < len(ts) else ts[j]
                            for j in range(0, len(ts), 2)
                        ]
                    tr_v[pl.ds(tb + r * 16, 16)] = ts[0]
                gs = [plsc.load_gather(tr_v, [gid]) for gid in gather_ids[tb]]
                while len(gs) > 1:
                    gs = [
                        gs[j] + gs[j + 1] if j + 1 < len(gs) else gs[j]
                        for j in range(0, len(gs), 2)
                    ]
                sims_v[s][pl.ds(i * NCOL_OUT, NCOL_OUT)] = gs[0] * sign

            @pl.loop(0, BLK)
            def _(i):
                one_sample(i, 0)

            wb_copy(g, s).start()

        for g in range(min(NSLOT, nblk)):
            stage_a_idx(g, g % NSLOT)
            stage_a_cp(g, g % NSLOT)
        for g in range(min(NSLOT - 1, nblk)):
            stage_b(g, g % NSLOT)

        @pl.loop(0, nblk, step=NSLOT)
        def _(g0):
            for dg in range(NSLOT):
                g = g0 + dg
                s = dg

                @pl.when(g + NSLOT - 1 < nblk)
                def _():
                    stage_b(g + NSLOT - 1, (dg + NSLOT - 1) % NSLOT)

                @pl.when(g < nblk)
                def _():
                    stage_c_wait(g, s)

                @pl.when(g + NSLOT < nblk)
                def _():
                    stage_a_idx(g + NSLOT, s)

                @pl.when(g < nblk)
                def _():
                    stage_c_comp(g, s)

                @pl.when(g + NSLOT < nblk)
                def _():
                    stage_a_cp(g + NSLOT, s)

        for t in range(NSLOT):
            g = nblk - NSLOT + t
            wb_copy(g, g % NSLOT).wait()

    return sc_kernel(z_flat, cp_all, idx_flat)


def _tc_reduce(sims2d, w2d):
    rows = sims2d.shape[0]
    br = 2048
    grid = rows // br

    def body(a_ref, w_ref, o_ref):
        @pl.when(pl.program_id(0) == 0)
        def _():
            o_ref[...] = jnp.zeros_like(o_ref)

        x = a_ref[...]
        sp = jnp.maximum(x, 0.0) + jnp.log1p(jnp.exp(-jnp.abs(x)))
        o_ref[...] += jnp.sum(w_ref[...] * sp).reshape(1, 1)

    out = pl.pallas_call(
        body,
        grid=(grid,),
        in_specs=[
            pl.BlockSpec((br, 128), lambda i: (i, 0)),
            pl.BlockSpec((br, 128), lambda i: (i, 0)),
        ],
        out_specs=pl.BlockSpec((1, 1), lambda i: (0, 0)),
        out_shape=jax.ShapeDtypeStruct((1, 1), jnp.float32),
    )(sims2d, w2d)
    return out[0, 0]


def kernel(z, c, predictions):
    del c
    B, C, S = z.shape
    K = predictions.shape[0]
    idx_flat, w_flat, P_pad = _constants(B, C, S, K)

    z_flat = jnp.transpose(z, (0, 2, 1)).reshape(-1, C).astype(jnp.bfloat16)
    z_flat = jax.lax.bitcast_convert_type(
        z_flat.reshape(-1, C // 2, 2), jnp.float32
    )
    cp_all = (
        jnp.transpose(predictions, (0, 1, 3, 2)).reshape(-1, C)
        * jnp.float32(1.0 / TEMP)
    ).astype(jnp.bfloat16)

    sims = _sc_sims(z_flat, cp_all, jnp.asarray(idx_flat), P_pad)
    sims2d = sims.reshape(-1, 128)
    w2d = jnp.asarray(w_flat).reshape(-1, 128)
    return _tc_reduce(sims2d, w2d)


# speedup vs baseline: 1.0336x; 1.0278x over previous
import functools

import jax
import jax.numpy as jnp
import numpy as np
from jax import lax
from jax.experimental import pallas as pl
from jax.experimental.pallas import tpu as pltpu
from jax.experimental.pallas import tpu_sc as plsc

NUM_NEG = 10
NCOL = NUM_NEG + 1
NCOL_OUT = 16
TEMP = 0.1

NC = 2
NS = 16
NW = NC * NS

BLK = 32
NSLOT = 3


def _sc_worker_id():
    return lax.axis_index("s") * NC + lax.axis_index("c")


def _tf2x32(k0, k1, x0, x1):
    rot0, rot1 = (13, 15, 26, 6), (17, 29, 16, 24)
    u32 = np.uint32
    ks0, ks1 = u32(k0), u32(k1)
    ks2 = ks0 ^ ks1 ^ u32(0x1BD11BDA)
    x0 = (x0 + ks0).astype(u32)
    x1 = (x1 + ks1).astype(u32)

    def rounds(x0, x1, rots):
        for r in rots:
            x0 = (x0 + x1).astype(u32)
            x1 = ((x1 << u32(r)) | (x1 >> u32(32 - r))).astype(u32)
            x1 = x0 ^ x1
        return x0, x1

    for i, (rots, ka, kb) in enumerate(
        [(rot0, ks1, ks2), (rot1, ks2, ks0), (rot0, ks0, ks1),
         (rot1, ks1, ks2), (rot0, ks2, ks0)]
    ):
        x0, x1 = rounds(x0, x1, rots)
        x0 = (x0 + ka).astype(u32)
        x1 = (x1 + kb + u32(i + 1)).astype(u32)
    return x0, x1


def _np_randint(key01, k, n, m, span):
    u32 = np.uint32
    k0, k1 = key01
    f0, f1 = _tf2x32(k0, k1, np.array([0], u32), np.array([k], u32))
    kf0, kf1 = f0[0], f1[0]
    b1, b2 = _tf2x32(kf0, kf1, np.zeros(2, u32), np.arange(2, dtype=u32))
    sub = [(b1[0], b2[0]), (b1[1], b2[1])]
    size = n * m
    i = np.arange(size, dtype=np.uint64)
    c1 = (i >> np.uint64(32)).astype(u32)
    c2 = i.astype(u32)
    o0, o1 = _tf2x32(sub[0][0], sub[0][1], c1, c2)
    hi = (o0 ^ o1).reshape(n, m)
    o0, o1 = _tf2x32(sub[1][0], sub[1][1], c1, c2)
    lo = (o0 ^ o1).reshape(n, m)
    span_u = u32(span)
    mult = u32(np.uint64(65536) % np.uint64(span))
    mult = u32((np.uint64(mult) * np.uint64(mult)) % np.uint64(span))
    val = ((hi % span_u) * mult + (lo % span_u)) % span_u
    return val.astype(np.int32)


@functools.lru_cache(maxsize=None)
def _constants(B, C, S, K):
    idx = np.zeros((K, B, S, NCOL), np.int32)
    w = np.zeros((K, B, S, NCOL_OUT), np.float32)
    for k in range(1, K + 1):
        L = S - k
        npos = B * L
        b = np.arange(B, dtype=np.int64)[:, None]
        i = np.arange(L, dtype=np.int64)[None, :]
        idx[k - 1, :, :L, 0] = (b * S + i + k).astype(np.int32)
        idx[k - 1, :, :L, 1:] = _np_randint(
            (np.uint32(0), np.uint32(1234)), k, npos, NUM_NEG, B * S
        ).reshape(B, L, NUM_NEG)
        w[k - 1, :, :L, 0] = 1.0 / (K * npos)
        w[k - 1, :, :L, 1:NCOL] = 1.0 / (K * npos * NUM_NEG)
    P_pad = K * B * S
    assert P_pad % (NW * BLK) == 0
    return idx.reshape(-1), w.reshape(-1), P_pad


def _sc_sims(z_flat, cp_all, idx_flat, P_pad):
    samp_per_tile = P_pad // NW
    nblk = samp_per_tile // BLK
    ipb = BLK * NCOL
    C = cp_all.shape[1]
    zw = z_flat.shape[1]
    nc32 = C // 32
    chunks = []
    off = 0
    while off < ipb:
        n = min(128, ipb - off)
        chunks.append((off, n))
        off += n

    mesh = plsc.VectorSubcoreMesh(
        core_axis_name="c", subcore_axis_name="s", num_cores=NC, num_subcores=NS
    )

    @functools.partial(
        pl.kernel,
        out_type=jax.ShapeDtypeStruct((P_pad * NCOL_OUT,), jnp.float32),
        mesh=mesh,
        compiler_params=pltpu.CompilerParams(
            needs_layout_passes=False, use_tc_tiling_on_sc=False
        ),
        scratch_types=dict(
            idx_v=[[pltpu.VMEM((n,), jnp.int32) for _, n in chunks] for _ in range(NSLOT)],
            z_v=[pltpu.VMEM((ipb, zw), jnp.float32) for _ in range(NSLOT)],
            cp_v=[pltpu.VMEM((BLK, C), jnp.bfloat16) for _ in range(NSLOT)],
            sims_v=[pltpu.VMEM((BLK * NCOL_OUT,), jnp.float32) for _ in range(NSLOT)],
            tr_v=pltpu.VMEM((2 * NCOL_OUT * 16,), jnp.float32),
            sem_idx=[pltpu.SemaphoreType.DMA for _ in range(NSLOT)],
            sem_g=[pltpu.SemaphoreType.DMA for _ in range(NSLOT)],
            sem_cp=[pltpu.SemaphoreType.DMA for _ in range(NSLOT)],
            sem_wb=[pltpu.SemaphoreType.DMA for _ in range(NSLOT)],
        ),
    )
    def sc_kernel(z_hbm, cp_hbm, idx_hbm, out_hbm, *, idx_v, z_v, cp_v, sims_v,
                  tr_v, sem_idx, sem_g, sem_cp, sem_wb):
        wid = _sc_worker_id()
        samp0 = wid * samp_per_tile

        for tb in (0, NCOL_OUT * 16):
            for r in range(NCOL, NCOL_OUT):
                tr_v[pl.ds(tb + r * 16, 16)] = jnp.zeros((16,), jnp.float32)

        def idx_copies(g, s):
            return [
                pltpu.make_async_copy(
                    idx_hbm.at[pl.ds((samp0 + g * BLK) * NCOL + off, n)],
                    idx_v[s][ci],
                    sem_idx[s],
                )
                for ci, (off, n) in enumerate(chunks)
            ]

        def cp_copy(g, s):
            return pltpu.make_async_copy(
                cp_hbm.at[pl.ds(samp0 + g * BLK, BLK)], cp_v[s], sem_cp[s]
            )

        def gather_copies(s):
            return [
                pltpu.make_async_copy(
                    z_hbm.at[idx_v[s][ci]],
                    z_v[s].at[pl.ds(off, n)],
                    sem_g[s],
                )
                for ci, (off, n) in enumerate(chunks)
            ]

        def wb_copy(g, s):
            return pltpu.make_async_copy(
                sims_v[s],
                out_hbm.at[pl.ds((samp0 + g * BLK) * NCOL_OUT, BLK * NCOL_OUT)],
                sem_wb[s],
            )

        def stage_a_idx(g, s):
            for cpy in idx_copies(g, s):
                cpy.start()

        def stage_a_cp(g, s):
            cp_copy(g, s).start()

        def stage_b(g, s):
            for cpy in idx_copies(g, s):
                cpy.wait()
            for cpy in gather_copies(s):
                cpy.start()

        def stage_c_wait(g, s):
            for cpy in gather_copies(s):
                cpy.wait()
            cp_copy(g, s).wait()

            @pl.when(g >= NSLOT)
            def _():
                wb_copy(g - NSLOT, s).wait()

        def stage_c_comp(g, s):

            lanes = lax.iota(jnp.int32, 16)
            sign = jnp.where(lanes == 0, -1.0, 1.0).astype(jnp.float32)
            gather_ids = {
                tb: [lanes * 16 + (tb + c) for c in range(16)]
                for tb in (0, NCOL_OUT * 16)
            }

            def one_sample(i, tb):
                cpv = []
                for c32 in range(nc32):
                    cpv.extend(
                        plsc.unpack(
                            cp_v[s][i, pl.ds(c32 * 32, 32)],
                            format=plsc.PackFormat.INTERLEAVED,
                            preferred_element_type=jnp.float32,
                        )
                    )
                for r in range(NCOL):
                    row = i * NCOL + r
                    ts = []
                    for c32 in range(nc32):
                        za, zb = plsc.unpack(
                            plsc.bitcast(
                                z_v[s][row, pl.ds(c32 * 16, 16)], jnp.bfloat16
                            ),
                            format=plsc.PackFormat.INTERLEAVED,
                            preferred_element_type=jnp.float32,
                        )
                        ts.append(za * cpv[2 * c32] + zb * cpv[2 * c32 + 1])
                    acc = ts[0]
                    for t in ts[1:]:
                        acc = acc + t
                    tr_v[pl.ds(tb + r * 16, 16)] = acc
                res = plsc.load_gather(tr_v, [gather_ids[tb][0]])
                for gid in gather_ids[tb][1:]:
                    res = res + plsc.load_gather(tr_v, [gid])
                sims_v[s][pl.ds(i * NCOL_OUT, NCOL_OUT)] = res * sign

            @pl.loop(0, BLK)
            def _(i):
                one_sample(i, 0)

            wb_copy(g, s).start()

        for g in range(min(NSLOT, nblk)):
            stage_a_idx(g, g % NSLOT)
            stage_a_cp(g, g % NSLOT)
        for g in range(min(NSLOT - 1, nblk)):
            stage_b(g, g % NSLOT)

        @pl.loop(0, nblk, step=NSLOT)
        def _(g0):
            for dg in range(NSLOT):
                g = g0 + dg
                s = dg

                @pl.when(g + NSLOT - 1 < nblk)
                def _():
                    stage_b(g + NSLOT - 1, (dg + NSLOT - 1) % NSLOT)

                @pl.when(g < nblk)
                def _():
                    stage_c_wait(g, s)

                @pl.when(g + NSLOT < nblk)
                def _():
                    stage_a_idx(g + NSLOT, s)

                @pl.when(g < nblk)
                def _():
                    stage_c_comp(g, s)

                @pl.when(g + NSLOT < nblk)
                def _():
                    stage_a_cp(g + NSLOT, s)

        for t in range(NSLOT):
            g = nblk - NSLOT + t
            wb_copy(g, g % NSLOT).wait()

    return sc_kernel(z_flat, cp_all, idx_flat)


def _tc_reduce(sims2d, w2d):
    rows = sims2d.shape[0]
    br = 2048
    grid = rows // br

    def body(a_ref, w_ref, o_ref):
        @pl.when(pl.program_id(0) == 0)
        def _():
            o_ref[...] = jnp.zeros_like(o_ref)

        x = a_ref[...]
        sp = jnp.maximum(x, 0.0) + jnp.log1p(jnp.exp(-jnp.abs(x)))
        o_ref[...] += jnp.sum(w_ref[...] * sp).reshape(1, 1)

    out = pl.pallas_call(
        body,
        grid=(grid,),
        in_specs=[
            pl.BlockSpec((br, 128), lambda i: (i, 0)),
            pl.BlockSpec((br, 128), lambda i: (i, 0)),
        ],
        out_specs=pl.BlockSpec((1, 1), lambda i: (0, 0)),
        out_shape=jax.ShapeDtypeStruct((1, 1), jnp.float32),
    )(sims2d, w2d)
    return out[0, 0]


def kernel(z, c, predictions):
    del c
    B, C, S = z.shape
    K = predictions.shape[0]
    idx_flat, w_flat, P_pad = _constants(B, C, S, K)

    z_flat = jnp.transpose(z, (0, 2, 1)).reshape(-1, C).astype(jnp.bfloat16)
    z_flat = jax.lax.bitcast_convert_type(
        z_flat.reshape(-1, C // 2, 2), jnp.float32
    )
    cp_all = (
        jnp.transpose(predictions, (0, 1, 3, 2)).reshape(-1, C)
        * jnp.float32(1.0 / TEMP)
    ).astype(jnp.bfloat16)

    sims = _sc_sims(z_flat, cp_all, jnp.asarray(idx_flat), P_pad)
    sims2d = sims.reshape(-1, 128)
    w2d = jnp.asarray(w_flat).reshape(-1, 128)
    return _tc_reduce(sims2d, w2d)
